# Initial kernel scaffold; baseline (speedup 1.0000x reference)
import jax
import jax.numpy as jnp
from jax.experimental import pallas as pl


def kernel(outputs_pred_logits, outputs_pred_boxes, target_sizes, image_names):
    raise NotImplementedError("write your pallas kernel here")


# speedup vs baseline: 5.6563x; 5.6563x over previous
import functools

import jax
import jax.numpy as jnp
from jax import lax
from jax.experimental import pallas as pl
from jax.experimental.pallas import tpu as pltpu
from jax.experimental.pallas import tpu_sc as plsc

B = 64
Q = 900
C = 91
K = 300
QC = Q * C
L = 16
NV = (QC + L - 1) // L
QCP = NV * L
NC, NS = 2, 16
NW = NC * NS
BPW = B // NW
SEL = 320
KP = 304
HB = 1024
IDX_PAD = 0x7FFFFFF0


def _take(v, idx):
    return lax.gather(
        v, idx[:, None],
        lax.GatherDimensionNumbers(
            offset_dims=(), collapsed_slice_dims=(0,), start_index_map=(0,)),
        (1,), mode=lax.GatherScatterMode.PROMISE_IN_BOUNDS)


def _splat(x, dtype=jnp.int32):
    return jnp.broadcast_to(jnp.asarray(x, dtype), (L,))


def _sc_body(prob_hbm, boxes_hbm, scale_hbm,
             scores_hbm, labels_hbm, boxes_out_hbm,
             p_v, boxes_v, scale_v, hist, sel_val, sel_idx,
             oscore, olabel, oboxid, oboxes):
    cid = lax.axis_index("c")
    sid = lax.axis_index("s")
    wid = sid * NC + cid

    lanes = lax.iota(jnp.int32, L)
    ones = jnp.ones((L,), jnp.int32)
    zeros_f = jnp.zeros((L,), jnp.float32)
    rep4 = lanes >> 2
    mod4 = lanes & 3
    idx_cxy = (rep4 << 2) + (lanes & 1)
    idx_wh = idx_cxy + 2
    coef = jnp.where((lanes & 2) == 0, -0.5, 0.5).astype(jnp.float32)

    def load_p(i):
        v = jnp.maximum(p_v[pl.ds(i * L, L)], 0.0)
        u = lax.bitcast_convert_type(v, jnp.int32)
        gidx = _splat(i * L) + lanes
        return v, u, gidx, gidx < QC

    def hist_pass(shift, pfx, match_shift):
        def zero_body(t, _):
            hist[pl.ds(t * L, L)] = jnp.zeros((L,), jnp.int32)
            return 0
        lax.fori_loop(0, HB, zero_body, 0)

        def body(i, _):
            v, u, gidx, valid = load_p(i)
            d = lax.shift_right_logical(u, shift) & (HB - 1)
            m = valid
            if match_shift is not None:
                m = m & (lax.shift_right_logical(u, match_shift)
                         == _splat(pfx))
            plsc.addupdate_scatter(hist, [d * L + lanes], ones, mask=m)
            return 0
        lax.fori_loop(0, NV, body, 0)

    def hist_scan(need):
        def body(t, carry):
            cum, dsel, above, found = carry
            r = HB - 1 - t
            tot = jnp.sum(hist[pl.ds(r * L, L)])
            hit = jnp.logical_and(jnp.logical_not(found), cum + tot >= need)
            dsel = jnp.where(hit, r, dsel)
            above = jnp.where(hit, cum, above)
            found = jnp.logical_or(found, hit)
            return cum + tot, dsel, above, found
        _, dsel, above, _ = lax.fori_loop(
            0, HB, body,
            (jnp.int32(0), jnp.int32(0), jnp.int32(0), jnp.bool_(False)))
        return dsel, above

    def process(b):
        pltpu.sync_copy(prob_hbm.at[b], p_v)
        pltpu.sync_copy(boxes_hbm.at[b], boxes_v)
        pltpu.sync_copy(scale_hbm.at[b], scale_v)

        hist_pass(20, None, None)
        b1, a1 = hist_scan(jnp.int32(K))
        need = jnp.int32(K) - a1
        hist_pass(10, b1, 20)
        b2, a2 = hist_scan(need)
        need = need - a2
        pfx2 = (b1 << 10) | b2
        hist_pass(0, pfx2, 10)
        b3, a3 = hist_scan(need)
        need = need - a3
        t_bits = (pfx2 << 10) | b3

        def init_body(j, _):
            sel_val[pl.ds(j * L, L)] = jnp.full((L,), -1.0, jnp.float32)
            sel_idx[pl.ds(j * L, L)] = jnp.full((L,), IDX_PAD, jnp.int32)
            return 0
        lax.fori_loop(0, SEL // L, init_body, 0)

        def zero_id(j, _):
            oboxid[pl.ds(j * L, L)] = jnp.zeros((L,), jnp.int32)
            return 0
        lax.fori_loop(0, KP // L, zero_id, 0)

        def collect(i, carry):
            off, taken = carry
            v, u, gidx, valid = load_p(i)
            m_gt = jnp.logical_and(u > t_bits, valid)
            m_eq = jnp.logical_and(jnp.logical_and(u == t_bits, valid),
                                   jnp.broadcast_to(taken < need, (L,)))
            m = jnp.logical_or(m_gt, m_eq)
            mi = m.astype(jnp.int32)
            pos = _splat(off) + plsc.cumsum(mi) - mi
            plsc.store_scatter(sel_val, [pos], v, mask=m)
            plsc.store_scatter(sel_idx, [pos], gidx, mask=m)
            return off + jnp.sum(mi), taken + jnp.sum(m_eq.astype(jnp.int32))
        lax.fori_loop(0, NV, collect, (jnp.int32(0), jnp.int32(0)))

        def rank_body(i, _):
            base = i - (i & (L - 1))
            lane = _splat(i & (L - 1))
            vi = _take(sel_val[pl.ds(base, L)], lane)
            ii = _take(sel_idx[pl.ds(base, L)], lane)
            rank = jnp.zeros((L,), jnp.int32)
            for j in range(SEL // L):
                va = sel_val[pl.ds(j * L, L)]
                ia = sel_idx[pl.ds(j * L, L)]
                m = jnp.logical_or(
                    va > vi,
                    jnp.logical_and(va == vi, ia < ii))
                rank = rank + plsc.all_reduce_population_count(m)
            mw = jnp.logical_and(lanes == 0, rank < K)
            plsc.store_scatter(oscore, [rank], vi, mask=mw)
            plsc.store_scatter(olabel, [rank], lax.rem(ii, _splat(C)),
                               mask=mw)
            plsc.store_scatter(oboxid, [rank], lax.div(ii, _splat(C)),
                               mask=mw)
            return 0
        lax.fori_loop(0, SEL, rank_body, 0)

        scale = scale_v[:]
        for vv in range(KP // 4):
            base = (4 * vv) & ~(L - 1)
            rows = _take(oboxid[pl.ds(base, L)], _splat(4 * vv - base) + rep4)
            bx = plsc.load_gather(boxes_v, [rows * 4 + mod4])
            out = (_take(bx, idx_cxy) + coef * _take(bx, idx_wh)) * scale
            oboxes[pl.ds(16 * vv, L)] = out

        pltpu.sync_copy(oscore, scores_hbm.at[b])
        pltpu.sync_copy(olabel, labels_hbm.at[b])
        pltpu.sync_copy(oboxes, boxes_out_hbm.at[b])

    for k in range(BPW):
        process(wid + NW * k)


@jax.jit
def _post_process_sc(prob_pad, boxes_flat, scale):
    mesh = plsc.VectorSubcoreMesh(core_axis_name="c", subcore_axis_name="s",
                                  num_cores=NC, num_subcores=NS)
    fn = pl.kernel(
        _sc_body,
        out_type=[
            jax.ShapeDtypeStruct((B, KP), jnp.float32),
            jax.ShapeDtypeStruct((B, KP), jnp.int32),
            jax.ShapeDtypeStruct((B, 4 * KP), jnp.float32),
        ],
        mesh=mesh,
        compiler_params=pltpu.CompilerParams(needs_layout_passes=False),
        scratch_types=[
            pltpu.VMEM((QCP,), jnp.float32),
            pltpu.VMEM((4 * Q,), jnp.float32),
            pltpu.VMEM((L,), jnp.float32),
            pltpu.VMEM((HB * L,), jnp.int32),
            pltpu.VMEM((SEL,), jnp.float32),
            pltpu.VMEM((SEL,), jnp.int32),
            pltpu.VMEM((KP,), jnp.float32),
            pltpu.VMEM((KP,), jnp.int32),
            pltpu.VMEM((KP,), jnp.int32),
            pltpu.VMEM((4 * KP,), jnp.float32),
        ],
    )
    return fn(prob_pad, boxes_flat, scale)


def kernel(outputs_pred_logits, outputs_pred_boxes, target_sizes, image_names):
    prob = jax.nn.sigmoid(outputs_pred_logits).reshape(B, QC)
    prob_pad = jnp.pad(prob, ((0, 0), (0, QCP - QC)), constant_values=-1.0)
    boxes_flat = outputs_pred_boxes.reshape(B, 4 * Q)
    img_h = target_sizes[:, 0].astype(jnp.float32)
    img_w = target_sizes[:, 1].astype(jnp.float32)
    scale = jnp.tile(jnp.stack([img_w, img_h, img_w, img_h], axis=1), (1, 4))
    scores_p, labels_p, boxes_p = _post_process_sc(prob_pad, boxes_flat, scale)
    scores = scores_p[:, :K]
    labels = labels_p[:, :K]
    boxes = boxes_p[:, :4 * K].reshape(B, K, 4)
    return scores, labels, boxes, image_names, target_sizes


# speedup vs baseline: 8.2168x; 1.4527x over previous
import jax
import jax.numpy as jnp
from jax import lax
from jax.experimental import pallas as pl
from jax.experimental.pallas import tpu as pltpu
from jax.experimental.pallas import tpu_sc as plsc

B = 64
Q = 900
C = 91
K = 300
QC = Q * C
L = 16
NV = (QC + L - 1) // L
QCP = NV * L
NC, NS = 2, 16
NW = NC * NS
BPW = B // NW
SEL = 320
KP = 304
HB = 1024
CAP = 6144
IDX_PAD = 0x7FFFFF00


def _take(v, idx):
    return lax.gather(
        v, idx[:, None],
        lax.GatherDimensionNumbers(
            offset_dims=(), collapsed_slice_dims=(0,), start_index_map=(0,)),
        (1,), mode=lax.GatherScatterMode.PROMISE_IN_BOUNDS)


def _splat(x, dtype=jnp.int32):
    return jnp.broadcast_to(jnp.asarray(x, dtype), (L,))


def _sc_body(prob_hbm, boxes_hbm, scale_hbm,
             scores_hbm, labels_hbm, boxes_out_hbm,
             p_v, boxes_v, scale_v, hist, tot_v, sel_val, sel_idx,
             cand_val, cand_idx, oscore, olabel, oboxid, oboxes):
    cid = lax.axis_index("c")
    sid = lax.axis_index("s")
    wid = sid * NC + cid

    lanes = lax.iota(jnp.int32, L)
    ones = jnp.ones((L,), jnp.int32)
    lane_h = lanes * HB
    lane_r = lanes * 32
    rep4 = lanes >> 2
    mod4 = lanes & 3
    idx_cxy = (rep4 << 2) + (lanes & 1)
    idx_wh = idx_cxy + 2
    coef = jnp.where((lanes & 2) == 0, -0.5, 0.5).astype(jnp.float32)

    def load_p(i):
        v = jnp.maximum(p_v[pl.ds(i * L, L)], 0.0)
        u = lax.bitcast_convert_type(v, jnp.int32)
        gidx = _splat(i * L) + lanes
        return v, u, gidx, gidx < QC

    def in_group_pick(acc, needg):
        rev = lax.rev(acc, (0,))
        cs = plsc.cumsum(rev)
        m = cs >= _splat(needg)
        k = jnp.broadcast_to(plsc.all_reduce_ffs(m), (L,))
        sel = lanes == k
        k_s = jnp.sum(jnp.where(sel, lanes, 0))
        abv = jnp.sum(jnp.where(sel, cs - rev, 0))
        return k_s, abv

    def scan1024(need):
        def g_body(t, carry):
            cum, gsel, above, found = carry
            g = 63 - t
            acc = hist[pl.ds(16 * g, L)]
            for l in range(1, L):
                acc = acc + hist[pl.ds(l * HB + 16 * g, L)]
            tot_v[pl.ds(16 * g, L)] = acc
            tg = jnp.sum(acc)
            hit = jnp.logical_and(jnp.logical_not(found), cum + tg >= need)
            gsel = jnp.where(hit, g, gsel)
            above = jnp.where(hit, cum, above)
            found = jnp.logical_or(found, hit)
            return cum + tg, gsel, above, found
        _, gsel, above, _ = lax.fori_loop(
            0, 64, g_body,
            (jnp.int32(0), jnp.int32(0), jnp.int32(0), jnp.bool_(False)))
        acc = tot_v[pl.ds(16 * gsel, L)]
        k_s, abv_g = in_group_pick(acc, need - above)
        bucket = 16 * gsel + 15 - k_s
        above = above + abv_g
        cnt = jnp.sum(jnp.where(lanes == jnp.broadcast_to(k_s, (L,)),
                                lax.rev(acc, (0,)), 0))
        return bucket, above, cnt

    def scan32(need):
        t0 = hist[pl.ds(0, L)]
        t1 = hist[pl.ds(16, L)]
        for l in range(1, L):
            t0 = t0 + hist[pl.ds(l * 32, L)]
            t1 = t1 + hist[pl.ds(l * 32 + 16, L)]
        c1 = jnp.sum(t1)
        in_hi = need <= c1
        acc = jnp.where(_splat(in_hi, jnp.bool_), t1, t0)
        needg = jnp.where(in_hi, need, need - c1)
        k_s, abv_g = in_group_pick(acc, needg)
        bucket = jnp.where(in_hi, 16, 0) + 15 - k_s
        above = jnp.where(in_hi, jnp.int32(0), c1) + abv_g
        return bucket, above

    def zero_hist(n_vregs):
        def zbody(t, _):
            hist[pl.ds(t * L, L)] = jnp.zeros((L,), jnp.int32)
            return 0
        lax.fori_loop(0, n_vregs, zbody, 0)

    def append(buf_v, buf_i, off, m, v, gidx):
        mi = m.astype(jnp.int32)
        pos = off + plsc.cumsum(mi) - mi
        plsc.store_scatter(buf_v, [pos], v, mask=m)
        plsc.store_scatter(buf_i, [pos], gidx, mask=m)
        return off + plsc.all_reduce_population_count(m)

    def process(b):
        pltpu.sync_copy(prob_hbm.at[b], p_v)
        pltpu.sync_copy(boxes_hbm.at[b], boxes_v)
        pltpu.sync_copy(scale_hbm.at[b], scale_v)

        zero_hist(HB)

        def hist1(i, _):
            v, u, gidx, valid = load_p(i)
            d = lax.shift_right_logical(u, 20)
            plsc.addupdate_scatter(hist, [lane_h + d], ones, mask=valid)
            return 0
        lax.fori_loop(0, NV, hist1, 0)
        b1, a1, cnt_b1 = scan1024(jnp.int32(K))
        need1 = jnp.int32(K) - a1

        def init_body(j, _):
            sel_val[pl.ds(j * L, L)] = jnp.full((L,), -1.0, jnp.float32)
            sel_idx[pl.ds(j * L, L)] = jnp.full((L,), IDX_PAD, jnp.int32)
            return 0
        lax.fori_loop(0, SEL // L, init_body, 0)

        def zero_id(j, _):
            oboxid[pl.ds(j * L, L)] = jnp.zeros((L,), jnp.int32)
            return 0
        lax.fori_loop(0, KP // L, zero_id, 0)

        def rounds_5bit(load_fn, nv, pfx0, need0):
            pfx, nd = pfx0, need0
            for shift in (15, 10, 5, 0):
                zero_hist(32)

                def hbody(i, _, shift=shift, pfx=pfx):
                    v, u, valid = load_fn(i)
                    m = jnp.logical_and(
                        valid,
                        lax.shift_right_logical(u, shift + 5) == _splat(pfx))
                    d = lax.shift_right_logical(u, shift) & 31
                    plsc.addupdate_scatter(hist, [lane_r + d], ones, mask=m)
                    return 0
                lax.fori_loop(0, nv, hbody, 0)
                dsel, abv = scan32(nd)
                pfx = (pfx << 5) | dsel
                nd = nd - abv
            return pfx, nd

        def fast_path():
            def pass2(i, carry):
                off_s, off_c = carry
                v, u, gidx, valid = load_p(i)
                d = lax.shift_right_logical(u, 20)
                m_gt = jnp.logical_and(d > b1, valid)
                m_c = jnp.logical_and(d == b1, valid)
                off_s = append(sel_val, sel_idx, off_s, m_gt, v, gidx)
                off_c = append(cand_val, cand_idx, off_c, m_c, v, gidx)
                return off_s, off_c
            lax.fori_loop(0, NV, pass2, (_splat(0), _splat(0)))

            ncv = lax.div(cnt_b1 + (L - 1), jnp.int32(L))

            def load_c(i):
                v = cand_val[pl.ds(i * L, L)]
                u = lax.bitcast_convert_type(v, jnp.int32)
                valid = (_splat(i * L) + lanes) < _splat(cnt_b1)
                return v, u, valid

            t_bits, need = rounds_5bit(load_c, ncv, b1, need1)

            def cbody(i, carry):
                off, taken = carry
                v, u, valid = load_c(i)
                gidx = cand_idx[pl.ds(i * L, L)]
                m_gt = jnp.logical_and(u > t_bits, valid)
                m_eq = jnp.logical_and(
                    jnp.logical_and(u == t_bits, valid),
                    taken < _splat(need))
                off = append(sel_val, sel_idx,
                             off, jnp.logical_or(m_gt, m_eq), v, gidx)
                taken = taken + plsc.all_reduce_population_count(m_eq)
                return off, taken
            lax.fori_loop(0, ncv, cbody, (_splat(a1), _splat(0)))
            return jnp.int32(0)

        def slow_path():
            def load_f(i):
                v, u, gidx, valid = load_p(i)
                return v, u, valid

            t_bits, need = rounds_5bit(load_f, jnp.int32(NV), b1, need1)

            def cbody(i, carry):
                off, taken = carry
                v, u, gidx, valid = load_p(i)
                m_gt = jnp.logical_and(u > t_bits, valid)
                m_eq = jnp.logical_and(
                    jnp.logical_and(u == t_bits, valid),
                    taken < _splat(need))
                off = append(sel_val, sel_idx,
                             off, jnp.logical_or(m_gt, m_eq), v, gidx)
                taken = taken + plsc.all_reduce_population_count(m_eq)
                return off, taken
            lax.fori_loop(0, NV, cbody, (_splat(0), _splat(0)))
            return jnp.int32(0)

        lax.cond(cnt_b1 <= CAP, fast_path, slow_path)

        def rank_body(i, _):
            base = i - (i & (L - 1))
            lane = _splat(i & (L - 1))
            vi = _take(sel_val[pl.ds(base, L)], lane)
            ii = _take(sel_idx[pl.ds(base, L)], lane)
            rank = jnp.zeros((L,), jnp.int32)
            for j in range(SEL // L):
                va = sel_val[pl.ds(j * L, L)]
                ia = sel_idx[pl.ds(j * L, L)]
                m = jnp.logical_or(
                    va > vi,
                    jnp.logical_and(va == vi, ia < ii))
                rank = rank + plsc.all_reduce_population_count(m)
            mw = jnp.logical_and(lanes == 0, rank < K)
            plsc.store_scatter(oscore, [rank], vi, mask=mw)
            plsc.store_scatter(olabel, [rank], lax.rem(ii, _splat(C)),
                               mask=mw)
            plsc.store_scatter(oboxid, [rank], lax.div(ii, _splat(C)),
                               mask=mw)
            return 0
        lax.fori_loop(0, SEL, rank_body, 0)

        scale = scale_v[:]
        for vv in range(KP // 4):
            base = (4 * vv) & ~(L - 1)
            rows = _take(oboxid[pl.ds(base, L)], _splat(4 * vv - base) + rep4)
            bx = plsc.load_gather(boxes_v, [rows * 4 + mod4])
            out = (_take(bx, idx_cxy) + coef * _take(bx, idx_wh)) * scale
            oboxes[pl.ds(16 * vv, L)] = out

        pltpu.sync_copy(oscore, scores_hbm.at[b])
        pltpu.sync_copy(olabel, labels_hbm.at[b])
        pltpu.sync_copy(oboxes, boxes_out_hbm.at[b])

    for k in range(BPW):
        process(wid + NW * k)


@jax.jit
def _post_process_sc(prob, boxes_flat, scale):
    mesh = plsc.VectorSubcoreMesh(core_axis_name="c", subcore_axis_name="s",
                                  num_cores=NC, num_subcores=NS)
    fn = pl.kernel(
        _sc_body,
        out_type=[
            jax.ShapeDtypeStruct((B, KP), jnp.float32),
            jax.ShapeDtypeStruct((B, KP), jnp.int32),
            jax.ShapeDtypeStruct((B, 4 * KP), jnp.float32),
        ],
        mesh=mesh,
        compiler_params=pltpu.CompilerParams(needs_layout_passes=False),
        scratch_types=[
            pltpu.VMEM((QCP,), jnp.float32),
            pltpu.VMEM((4 * Q,), jnp.float32),
            pltpu.VMEM((L,), jnp.float32),
            pltpu.VMEM((HB * L,), jnp.int32),
            pltpu.VMEM((HB,), jnp.int32),
            pltpu.VMEM((SEL,), jnp.float32),
            pltpu.VMEM((SEL,), jnp.int32),
            pltpu.VMEM((CAP + L,), jnp.float32),
            pltpu.VMEM((CAP + L,), jnp.int32),
            pltpu.VMEM((KP,), jnp.float32),
            pltpu.VMEM((KP,), jnp.int32),
            pltpu.VMEM((KP,), jnp.int32),
            pltpu.VMEM((4 * KP,), jnp.float32),
        ],
    )
    return fn(prob, boxes_flat, scale)


def kernel(outputs_pred_logits, outputs_pred_boxes, target_sizes, image_names):
    prob = jax.nn.sigmoid(outputs_pred_logits).reshape(B, QC)
    prob = jnp.pad(prob, ((0, 0), (0, QCP - QC)), constant_values=-1.0)
    boxes_flat = outputs_pred_boxes.reshape(B, 4 * Q)
    img_h = target_sizes[:, 0].astype(jnp.float32)
    img_w = target_sizes[:, 1].astype(jnp.float32)
    scale = jnp.tile(jnp.stack([img_w, img_h, img_w, img_h], axis=1), (1, 4))
    scores_p, labels_p, boxes_p = _post_process_sc(prob, boxes_flat, scale)
    scores = scores_p[:, :K]
    labels = labels_p[:, :K]
    boxes = boxes_p[:, :4 * K].reshape(B, K, 4)
    return scores, labels, boxes, image_names, target_sizes


# speedup vs baseline: 9.7962x; 1.1922x over previous
import jax
import jax.numpy as jnp
from jax import lax
from jax.experimental import pallas as pl
from jax.experimental.pallas import tpu as pltpu
from jax.experimental.pallas import tpu_sc as plsc

B = 64
Q = 900
C = 91
K = 300
QC = Q * C
L = 16
NV = (QC + L - 1) // L
QCP = NV * L
NC, NS = 2, 16
NW = NC * NS
BPW = B // NW
SEL = 320
KP = 304
HB = 1024
CAP = 6144
IDX_PAD = 0x7FFFFF00


def _take(v, idx):
    return lax.gather(
        v, idx[:, None],
        lax.GatherDimensionNumbers(
            offset_dims=(), collapsed_slice_dims=(0,), start_index_map=(0,)),
        (1,), mode=lax.GatherScatterMode.PROMISE_IN_BOUNDS)


def _splat(x, dtype=jnp.int32):
    return jnp.broadcast_to(jnp.asarray(x, dtype), (L,))


def _sc_body(prob_hbm, boxes_hbm, scale_hbm,
             scores_hbm, labels_hbm, boxes_out_hbm,
             p_v, boxes_v, scale_v, hist, tot_v, sel_val, sel_idx,
             cand_val, cand_idx, oscore, olabel, oboxid, oboxes):
    cid = lax.axis_index("c")
    sid = lax.axis_index("s")
    wid = sid * NC + cid

    lanes = lax.iota(jnp.int32, L)
    ones = jnp.ones((L,), jnp.int32)
    lane_h = lanes * HB
    lane_r = lanes * 32
    rep4 = lanes >> 2
    mod4 = lanes & 3
    idx_cxy = (rep4 << 2) + (lanes & 1)
    idx_wh = idx_cxy + 2
    coef = jnp.where((lanes & 2) == 0, -0.5, 0.5).astype(jnp.float32)

    def load_p(i):
        v = jnp.maximum(p_v[pl.ds(i * L, L)], 0.0)
        u = lax.bitcast_convert_type(v, jnp.int32)
        gidx = _splat(i * L) + lanes
        return v, u, gidx, gidx < QC

    def in_group_pick(acc, needg):
        rev = lax.rev(acc, (0,))
        cs = plsc.cumsum(rev)
        m = cs >= _splat(needg)
        k = jnp.broadcast_to(plsc.all_reduce_ffs(m), (L,))
        sel = lanes == k
        k_s = jnp.sum(jnp.where(sel, lanes, 0))
        abv = jnp.sum(jnp.where(sel, cs - rev, 0))
        return k_s, abv

    def scan1024(need):
        @plsc.parallel_loop(
            0, 64, unroll=4,
            carry=(jnp.int32(0), jnp.int32(0), jnp.int32(0),
                   jnp.bool_(False)))
        def g_scan(t, carry):
            cum, gsel, above, found = carry
            g = 63 - t
            acc = hist[pl.ds(16 * g, L)]
            for l in range(1, L):
                acc = acc + hist[pl.ds(l * HB + 16 * g, L)]
            tot_v[pl.ds(16 * g, L)] = acc
            tg = jnp.sum(acc)
            hit = jnp.logical_and(jnp.logical_not(found), cum + tg >= need)
            gsel = jnp.where(hit, g, gsel)
            above = jnp.where(hit, cum, above)
            found = jnp.logical_or(found, hit)
            return cum + tg, gsel, above, found
        _, gsel, above, _ = g_scan
        acc = tot_v[pl.ds(16 * gsel, L)]
        k_s, abv_g = in_group_pick(acc, need - above)
        bucket = 16 * gsel + 15 - k_s
        above = above + abv_g
        cnt = jnp.sum(jnp.where(lanes == jnp.broadcast_to(k_s, (L,)),
                                lax.rev(acc, (0,)), 0))
        return bucket, above, cnt

    def scan32(need):
        t0 = hist[pl.ds(0, L)]
        t1 = hist[pl.ds(16, L)]
        for l in range(1, L):
            t0 = t0 + hist[pl.ds(l * 32, L)]
            t1 = t1 + hist[pl.ds(l * 32 + 16, L)]
        c1 = jnp.sum(t1)
        in_hi = need <= c1
        acc = jnp.where(_splat(in_hi, jnp.bool_), t1, t0)
        needg = jnp.where(in_hi, need, need - c1)
        k_s, abv_g = in_group_pick(acc, needg)
        bucket = jnp.where(in_hi, 16, 0) + 15 - k_s
        above = jnp.where(in_hi, jnp.int32(0), c1) + abv_g
        return bucket, above

    def zero_hist(n_vregs):
        @plsc.parallel_loop(0, n_vregs, unroll=8)
        def _(t):
            hist[pl.ds(t * L, L)] = jnp.zeros((L,), jnp.int32)

    def append(buf_v, buf_i, off, m, v, gidx):
        mi = m.astype(jnp.int32)
        pos = off + plsc.cumsum(mi) - mi
        plsc.store_scatter(buf_v, [pos], v, mask=m)
        plsc.store_scatter(buf_i, [pos], gidx, mask=m)
        return off + plsc.all_reduce_population_count(m)

    def process(b):
        pltpu.sync_copy(prob_hbm.at[b], p_v)
        pltpu.sync_copy(boxes_hbm.at[b], boxes_v)
        pltpu.sync_copy(scale_hbm.at[b], scale_v)

        zero_hist(HB)

        def hist1_one(i):
            v, u, gidx, valid = load_p(i)
            d = lax.shift_right_logical(u, 20)
            plsc.addupdate_scatter(hist, [lane_h + d], ones, mask=valid)

        def hist1(g, _):
            for t in range(4):
                hist1_one(g * 4 + t)
            return 0
        lax.fori_loop(0, NV // 4, hist1, 0)
        for i in range(NV - NV % 4, NV):
            hist1_one(jnp.int32(i))
        b1, a1, cnt_b1 = scan1024(jnp.int32(K))
        need1 = jnp.int32(K) - a1

        @plsc.parallel_loop(0, SEL // L, unroll=4)
        def _(j):
            sel_val[pl.ds(j * L, L)] = jnp.full((L,), -1.0, jnp.float32)
            sel_idx[pl.ds(j * L, L)] = jnp.full((L,), IDX_PAD, jnp.int32)

        @plsc.parallel_loop(0, KP // L, unroll=4)
        def _(j):
            oboxid[pl.ds(j * L, L)] = jnp.zeros((L,), jnp.int32)

        def rounds_5bit(load_fn, nv, pfx0, need0):
            pfx, nd = pfx0, need0
            for shift in (15, 10, 5, 0):
                zero_hist(32)

                def hbody(i, _, shift=shift, pfx=pfx):
                    v, u, valid = load_fn(i)
                    m = jnp.logical_and(
                        valid,
                        lax.shift_right_logical(u, shift + 5) == _splat(pfx))
                    d = lax.shift_right_logical(u, shift) & 31
                    plsc.addupdate_scatter(hist, [lane_r + d], ones, mask=m)
                    return 0
                lax.fori_loop(0, nv, hbody, 0)
                dsel, abv = scan32(nd)
                pfx = (pfx << 5) | dsel
                nd = nd - abv
            return pfx, nd

        def fast_path():
            @plsc.parallel_loop(0, NV, unroll=4, carry=(_splat(0), _splat(0)))
            def pass2(i, carry):
                off_s, off_c = carry
                v, u, gidx, valid = load_p(i)
                d = lax.shift_right_logical(u, 20)
                m_gt = jnp.logical_and(d > b1, valid)
                m_c = jnp.logical_and(d == b1, valid)
                off_s = append(sel_val, sel_idx, off_s, m_gt, v, gidx)
                off_c = append(cand_val, cand_idx, off_c, m_c, v, gidx)
                return off_s, off_c

            ncv = lax.div(cnt_b1 + (L - 1), jnp.int32(L))

            def load_c(i):
                v = cand_val[pl.ds(i * L, L)]
                u = lax.bitcast_convert_type(v, jnp.int32)
                valid = (_splat(i * L) + lanes) < _splat(cnt_b1)
                return v, u, valid

            t_bits, need = rounds_5bit(load_c, ncv, b1, need1)

            @plsc.parallel_loop(0, ncv, unroll=4,
                                carry=(_splat(a1), _splat(0)))
            def _(i, carry):
                off, taken = carry
                v, u, valid = load_c(i)
                gidx = cand_idx[pl.ds(i * L, L)]
                m_gt = jnp.logical_and(u > t_bits, valid)
                m_eq = jnp.logical_and(
                    jnp.logical_and(u == t_bits, valid),
                    taken < _splat(need))
                off = append(sel_val, sel_idx,
                             off, jnp.logical_or(m_gt, m_eq), v, gidx)
                taken = taken + plsc.all_reduce_population_count(m_eq)
                return off, taken
            return jnp.int32(0)

        def slow_path():
            def load_f(i):
                v, u, gidx, valid = load_p(i)
                return v, u, valid

            t_bits, need = rounds_5bit(load_f, jnp.int32(NV), b1, need1)

            @plsc.parallel_loop(0, NV, unroll=4,
                                carry=(_splat(0), _splat(0)))
            def _(i, carry):
                off, taken = carry
                v, u, gidx, valid = load_p(i)
                m_gt = jnp.logical_and(u > t_bits, valid)
                m_eq = jnp.logical_and(
                    jnp.logical_and(u == t_bits, valid),
                    taken < _splat(need))
                off = append(sel_val, sel_idx,
                             off, jnp.logical_or(m_gt, m_eq), v, gidx)
                taken = taken + plsc.all_reduce_population_count(m_eq)
                return off, taken
            return jnp.int32(0)

        lax.cond(cnt_b1 <= CAP, fast_path, slow_path)

        @plsc.parallel_loop(0, SEL, unroll=2)
        def _(i):
            base = i - (i & (L - 1))
            lane = _splat(i & (L - 1))
            vi = _take(sel_val[pl.ds(base, L)], lane)
            ii = _take(sel_idx[pl.ds(base, L)], lane)
            rank = jnp.zeros((L,), jnp.int32)
            for j in range(SEL // L):
                va = sel_val[pl.ds(j * L, L)]
                ia = sel_idx[pl.ds(j * L, L)]
                m = jnp.logical_or(
                    va > vi,
                    jnp.logical_and(va == vi, ia < ii))
                rank = rank + plsc.all_reduce_population_count(m)
            mw = jnp.logical_and(lanes == 0, rank < K)
            plsc.store_scatter(oscore, [rank], vi, mask=mw)
            plsc.store_scatter(olabel, [rank], lax.rem(ii, _splat(C)),
                               mask=mw)
            plsc.store_scatter(oboxid, [rank], lax.div(ii, _splat(C)),
                               mask=mw)

        scale = scale_v[:]
        for vv in range(KP // 4):
            base = (4 * vv) & ~(L - 1)
            rows = _take(oboxid[pl.ds(base, L)], _splat(4 * vv - base) + rep4)
            bx = plsc.load_gather(boxes_v, [rows * 4 + mod4])
            out = (_take(bx, idx_cxy) + coef * _take(bx, idx_wh)) * scale
            oboxes[pl.ds(16 * vv, L)] = out

        pltpu.sync_copy(oscore, scores_hbm.at[b])
        pltpu.sync_copy(olabel, labels_hbm.at[b])
        pltpu.sync_copy(oboxes, boxes_out_hbm.at[b])

    for k in range(BPW):
        process(wid + NW * k)


@jax.jit
def _post_process_sc(prob, boxes_flat, scale):
    mesh = plsc.VectorSubcoreMesh(core_axis_name="c", subcore_axis_name="s",
                                  num_cores=NC, num_subcores=NS)
    fn = pl.kernel(
        _sc_body,
        out_type=[
            jax.ShapeDtypeStruct((B, KP), jnp.float32),
            jax.ShapeDtypeStruct((B, KP), jnp.int32),
            jax.ShapeDtypeStruct((B, 4 * KP), jnp.float32),
        ],
        mesh=mesh,
        compiler_params=pltpu.CompilerParams(needs_layout_passes=False),
        scratch_types=[
            pltpu.VMEM((QCP,), jnp.float32),
            pltpu.VMEM((4 * Q,), jnp.float32),
            pltpu.VMEM((L,), jnp.float32),
            pltpu.VMEM((HB * L,), jnp.int32),
            pltpu.VMEM((HB,), jnp.int32),
            pltpu.VMEM((SEL,), jnp.float32),
            pltpu.VMEM((SEL,), jnp.int32),
            pltpu.VMEM((CAP + L,), jnp.float32),
            pltpu.VMEM((CAP + L,), jnp.int32),
            pltpu.VMEM((KP,), jnp.float32),
            pltpu.VMEM((KP,), jnp.int32),
            pltpu.VMEM((KP,), jnp.int32),
            pltpu.VMEM((4 * KP,), jnp.float32),
        ],
    )
    return fn(prob, boxes_flat, scale)


def kernel(outputs_pred_logits, outputs_pred_boxes, target_sizes, image_names):
    prob = jax.nn.sigmoid(outputs_pred_logits).reshape(B, QC)
    prob = jnp.pad(prob, ((0, 0), (0, QCP - QC)), constant_values=-1.0)
    boxes_flat = outputs_pred_boxes.reshape(B, 4 * Q)
    img_h = target_sizes[:, 0].astype(jnp.float32)
    img_w = target_sizes[:, 1].astype(jnp.float32)
    scale = jnp.tile(jnp.stack([img_w, img_h, img_w, img_h], axis=1), (1, 4))
    scores_p, labels_p, boxes_p = _post_process_sc(prob, boxes_flat, scale)
    scores = scores_p[:, :K]
    labels = labels_p[:, :K]
    boxes = boxes_p[:, :4 * K].reshape(B, K, 4)
    return scores, labels, boxes, image_names, target_sizes
